# baseline (speedup 1.0000x reference)
import functools

import jax
import jax.numpy as jnp
from jax import lax
from jax.experimental import pallas as pl
from jax.experimental.pallas import tpu as pltpu
from jax.experimental.pallas import tpu_sc as plsc

D = 128
K = 8
DD = D // K
M = 32
PAD = 8


def _seg_matrices(dtype=jnp.float32):
    lane = lax.broadcasted_iota(jnp.int32, (D, K), 0)
    cap = lax.broadcasted_iota(jnp.int32, (D, K), 1)
    s = (lane // DD == cap).astype(dtype)
    lane_t = lax.broadcasted_iota(jnp.int32, (K, D), 1)
    cap_t = lax.broadcasted_iota(jnp.int32, (K, D), 0)
    st = (lane_t // DD == cap_t).astype(dtype)
    return s, st


def _normalize_body(x_ref, o_ref, oi_ref):
    x = x_ref[...]
    n = x.shape[0]
    s, st = _seg_matrices()
    ss = lax.dot_general(x * x, s, (((1,), (0,)), ((), ())),
                         preferred_element_type=jnp.float32)
    den = jnp.maximum(jnp.sqrt(ss), 1e-12)
    inv = lax.dot_general(1.0 / den, st, (((1,), (0,)), ((), ())),
                          preferred_element_type=jnp.float32)
    xn = x * inv
    o_ref[...] = xn
    oi_ref[pl.ds(0, n), :] = lax.bitcast_convert_type(xn, jnp.int32)
    oi_ref[pl.ds(n, PAD), :] = jnp.zeros((PAD, D), jnp.int32)


def _unpack_bf16(zc):
    zlo = lax.bitcast_convert_type(zc << 16, jnp.float32)
    zhi = lax.bitcast_convert_type(zc & jnp.int32(-65536), jnp.float32)
    return jnp.concatenate([zlo, zhi], axis=1)


def _routing_body(mi_ref, z_ref, x_ref, o_ref):
    zb = _unpack_bf16(z_ref[...])
    xb = x_ref[...]
    b = xb.shape[0]
    s, st = _seg_matrices()

    def seg_sum(t):
        return lax.dot_general(t, s, (((1,), (0,)), ((), ())),
                               preferred_element_type=jnp.float32)

    def expand(t):
        return lax.dot_general(t, st, (((1,), (0,)), ((), ())),
                               preferred_element_type=jnp.float32)

    ones_kd = jnp.ones((K, D), jnp.float32)

    def body(_, u):
        ss = seg_sum(u * u)
        den = jnp.maximum(jnp.sqrt(ss), 1e-12)
        un = u * expand(1.0 / den)
        unb = jnp.broadcast_to(un.reshape(b, 1, D), (b, M, D)).reshape(b * M, D)
        logits = seg_sum(zb * unb)
        e = jnp.exp(logits)
        num = expand(e)
        dsum = lax.dot_general(e, ones_kd, (((1,), (0,)), ((), ())),
                               preferred_element_type=jnp.float32)
        w = zb * (num / dsum)
        return jnp.sum(w.reshape(b, M, D), axis=1) + xb

    u0 = jnp.zeros((b, D), jnp.float32)
    o_ref[...] = lax.fori_loop(0, mi_ref[0], body, u0)


def _make_gather(n_rows, e, kc):
    mesh = plsc.VectorSubcoreMesh(core_axis_name="c", subcore_axis_name="s")
    nw = 32
    ch = kc * 128
    nch = e // ch
    n_ss = pl.cdiv(nch, 2 * nw)
    hw = D // 2

    @functools.partial(
        pl.kernel, mesh=mesh,
        out_type=jax.ShapeDtypeStruct((e, hw), jnp.int32),
        scratch_types=[
            pltpu.VMEM((kc, 128), jnp.int32),
            pltpu.VMEM((kc, 128), jnp.int32),
            pltpu.VMEM((ch, D), jnp.int32),
            pltpu.VMEM((ch, D), jnp.int32),
            pltpu.VMEM((ch, hw), jnp.int32),
            pltpu.SemaphoreType.DMA,
            pltpu.SemaphoreType.DMA,
            pltpu.SemaphoreType.DMA,
        ],
    )
    def gather(table_hbm, nbr_hbm, out_hbm, idx0, idx1, rows0, rows1,
               pk, g0, g1, w):
        wid = lax.axis_index("s") * 2 + lax.axis_index("c")

        def fetch(c, idx_v, rows_v, g):
            for j in range(kc):
                pltpu.sync_copy(nbr_hbm.at[pl.ds(c * ch + j * 128, 128)],
                                idx_v.at[j])
            for j in range(kc):
                pltpu.async_copy(
                    table_hbm.at[idx_v.at[j]],
                    rows_v.at[pl.ds(j * 128, 128)], g)

        def fetch_wait(idx_v, rows_v, g):
            for j in range(kc):
                pltpu.make_async_copy(
                    table_hbm.at[idx_v.at[j]],
                    rows_v.at[pl.ds(j * 128, 128)], g).wait()

        def rne16(bits):
            return (bits + (((bits >> 16) & 1) + 0x7FFF)) >> 16

        def pack_rows(rows_v, pk_v):
            def prow(r, carry):
                for j in range(4):
                    a = rows_v[r, pl.ds(16 * j, 16)]
                    bb = rows_v[r, pl.ds(64 + 16 * j, 16)]
                    pk_v[r, pl.ds(16 * j, 16)] = (
                        (rne16(bb) << 16) | (rne16(a) & 0xFFFF))
                return carry
            lax.fori_loop(0, ch, prow, 0)

        def wb_start(c):
            pltpu.async_copy(pk, out_hbm.at[pl.ds(c * ch, ch)], w)

        def wb_wait(c):
            pltpu.make_async_copy(pk, out_hbm.at[pl.ds(c * ch, ch)], w).wait()

        def body(ss, carry):
            c0 = ss * 2 * nw + wid
            c1 = c0 + nw
            c1p = c0 - nw
            c0p = c0 - 2 * nw
            p1_valid = jnp.logical_and(c1p >= 0, c1p < nch)

            @pl.when(c0 < nch)
            def _():
                fetch(c0, idx0, rows0, g0)

            @pl.when(p1_valid)
            def _():
                fetch_wait(idx1, rows1, g1)

            @pl.when(jnp.logical_and(c0p >= 0, c0p < nch))
            def _():
                wb_wait(c0p)

            @pl.when(p1_valid)
            def _():
                pack_rows(rows1, pk)
                wb_start(c1p)

            @pl.when(c0 < nch)
            def _():
                fetch_wait(idx0, rows0, g0)

            @pl.when(c1 < nch)
            def _():
                fetch(c1, idx1, rows1, g1)

            @pl.when(p1_valid)
            def _():
                wb_wait(c1p)

            @pl.when(c0 < nch)
            def _():
                pack_rows(rows0, pk)
                wb_start(c0)

            return carry

        lax.fori_loop(0, n_ss + 1, body, 0)

    return gather


def kernel(x, neighbors, max_iter):
    n = x.shape[0]
    e = neighbors.shape[0]

    xn, xni = pl.pallas_call(
        _normalize_body,
        out_shape=(jax.ShapeDtypeStruct((n, D), jnp.float32),
                   jax.ShapeDtypeStruct((n + PAD, D), jnp.int32)),
    )(x)

    n_chunks = 5
    cn = n // n_chunks
    ce = cn * M
    blk = 400
    grid = cn // blk
    mi = jnp.reshape(jnp.asarray(max_iter, jnp.int32), (1,))
    gather = _make_gather(n + PAD, ce, 2)

    outs = []
    for i in range(n_chunks):
        nbr_i = lax.slice_in_dim(neighbors, i * ce, (i + 1) * ce)
        z_i = gather(xni, nbr_i)
        base = i * grid
        u_i = pl.pallas_call(
            _routing_body,
            grid=(grid,),
            in_specs=[
                pl.BlockSpec(memory_space=pltpu.SMEM),
                pl.BlockSpec((blk * M, D // 2), lambda j: (j, 0)),
                pl.BlockSpec((blk, D), lambda j, base=base: (base + j, 0)),
            ],
            out_specs=pl.BlockSpec((blk, D), lambda j: (j, 0)),
            out_shape=jax.ShapeDtypeStruct((cn, D), jnp.float32),
        )(mi, z_i, xn)
        outs.append(u_i)
    return jnp.concatenate(outs, axis=0)


# speedup vs baseline: 1.0003x; 1.0003x over previous
import functools

import jax
import jax.numpy as jnp
from jax import lax
from jax.experimental import pallas as pl
from jax.experimental.pallas import tpu as pltpu
from jax.experimental.pallas import tpu_sc as plsc

D = 128
K = 8
DD = D // K
M = 32
PAD = 8


def _seg_matrices(dtype=jnp.float32):
    lane = lax.broadcasted_iota(jnp.int32, (D, K), 0)
    cap = lax.broadcasted_iota(jnp.int32, (D, K), 1)
    s = (lane // DD == cap).astype(dtype)
    lane_t = lax.broadcasted_iota(jnp.int32, (K, D), 1)
    cap_t = lax.broadcasted_iota(jnp.int32, (K, D), 0)
    st = (lane_t // DD == cap_t).astype(dtype)
    return s, st


def _normalize_body(x_ref, o_ref, oi_ref):
    x = x_ref[...]
    n = x.shape[0]
    s, st = _seg_matrices()
    ss = lax.dot_general(x * x, s, (((1,), (0,)), ((), ())),
                         preferred_element_type=jnp.float32)
    den = jnp.maximum(jnp.sqrt(ss), 1e-12)
    inv = lax.dot_general(1.0 / den, st, (((1,), (0,)), ((), ())),
                          preferred_element_type=jnp.float32)
    xn = x * inv
    o_ref[...] = xn
    oi_ref[pl.ds(0, n), :] = lax.bitcast_convert_type(xn, jnp.int32)
    oi_ref[pl.ds(n, PAD), :] = jnp.zeros((PAD, D), jnp.int32)


def _unpack_bf16(zc):
    zlo = lax.bitcast_convert_type(zc << 16, jnp.float32)
    zhi = lax.bitcast_convert_type(zc & jnp.int32(-65536), jnp.float32)
    return jnp.concatenate([zlo, zhi], axis=1)


def _routing_body(mi_ref, z_ref, x_ref, o_ref):
    zb = _unpack_bf16(z_ref[...])
    xb = x_ref[...]
    b = xb.shape[0]
    s, st = _seg_matrices()

    def seg_sum(t):
        return lax.dot_general(t, s, (((1,), (0,)), ((), ())),
                               preferred_element_type=jnp.float32)

    def expand(t):
        return lax.dot_general(t, st, (((1,), (0,)), ((), ())),
                               preferred_element_type=jnp.float32)

    ones_kd = jnp.ones((K, D), jnp.float32)

    def body(_, u):
        ss = seg_sum(u * u)
        den = jnp.maximum(jnp.sqrt(ss), 1e-12)
        un = u * expand(1.0 / den)
        unb = jnp.broadcast_to(un.reshape(b, 1, D), (b, M, D)).reshape(b * M, D)
        logits = seg_sum(zb * unb)
        e = jnp.exp(logits)
        num = expand(e)
        dsum = lax.dot_general(e, ones_kd, (((1,), (0,)), ((), ())),
                               preferred_element_type=jnp.float32)
        w = zb * (num / dsum)
        return jnp.sum(w.reshape(b, M, D), axis=1) + xb

    u0 = jnp.zeros((b, D), jnp.float32)
    o_ref[...] = lax.fori_loop(0, mi_ref[0], body, u0)


def _make_gather(n_rows, e, kc):
    mesh = plsc.VectorSubcoreMesh(core_axis_name="c", subcore_axis_name="s")
    nw = 32
    ch = kc * 128
    nch = e // ch
    n_ss = pl.cdiv(nch, 2 * nw)
    hw = D // 2

    @functools.partial(
        pl.kernel, mesh=mesh,
        out_type=jax.ShapeDtypeStruct((e, hw), jnp.int32),
        scratch_types=[
            pltpu.VMEM((kc, 128), jnp.int32),
            pltpu.VMEM((kc, 128), jnp.int32),
            pltpu.VMEM((ch, D), jnp.int32),
            pltpu.VMEM((ch, D), jnp.int32),
            pltpu.VMEM((ch, hw), jnp.int32),
            pltpu.SemaphoreType.DMA,
            pltpu.SemaphoreType.DMA,
            pltpu.SemaphoreType.DMA,
        ],
    )
    def gather(table_hbm, nbr_hbm, out_hbm, idx0, idx1, rows0, rows1,
               pk, g0, g1, w):
        wid = lax.axis_index("s") * 2 + lax.axis_index("c")

        def fetch(c, idx_v, rows_v, g):
            for j in range(kc):
                pltpu.sync_copy(nbr_hbm.at[pl.ds(c * ch + j * 128, 128)],
                                idx_v.at[j])
            for j in range(kc):
                pltpu.async_copy(
                    table_hbm.at[idx_v.at[j]],
                    rows_v.at[pl.ds(j * 128, 128)], g)

        def fetch_wait(idx_v, rows_v, g):
            for j in range(kc):
                pltpu.make_async_copy(
                    table_hbm.at[idx_v.at[j]],
                    rows_v.at[pl.ds(j * 128, 128)], g).wait()

        def rne16(bits):
            return (bits + (((bits >> 16) & 1) + 0x7FFF)) >> 16

        def pack_rows(rows_v, pk_v):
            unr = 8

            def prow(t, carry):
                for dr in range(unr):
                    r = t * unr + dr
                    for j in range(4):
                        a = rows_v[r, pl.ds(16 * j, 16)]
                        bb = rows_v[r, pl.ds(64 + 16 * j, 16)]
                        pk_v[r, pl.ds(16 * j, 16)] = (
                            (rne16(bb) << 16) | (rne16(a) & 0xFFFF))
                return carry
            lax.fori_loop(0, ch // unr, prow, 0)

        def wb_start(c):
            pltpu.async_copy(pk, out_hbm.at[pl.ds(c * ch, ch)], w)

        def wb_wait(c):
            pltpu.make_async_copy(pk, out_hbm.at[pl.ds(c * ch, ch)], w).wait()

        def body(ss, carry):
            c0 = ss * 2 * nw + wid
            c1 = c0 + nw
            c1p = c0 - nw
            c0p = c0 - 2 * nw
            p1_valid = jnp.logical_and(c1p >= 0, c1p < nch)

            @pl.when(c0 < nch)
            def _():
                fetch(c0, idx0, rows0, g0)

            @pl.when(p1_valid)
            def _():
                fetch_wait(idx1, rows1, g1)

            @pl.when(jnp.logical_and(c0p >= 0, c0p < nch))
            def _():
                wb_wait(c0p)

            @pl.when(p1_valid)
            def _():
                pack_rows(rows1, pk)
                wb_start(c1p)

            @pl.when(c0 < nch)
            def _():
                fetch_wait(idx0, rows0, g0)

            @pl.when(c1 < nch)
            def _():
                fetch(c1, idx1, rows1, g1)

            @pl.when(p1_valid)
            def _():
                wb_wait(c1p)

            @pl.when(c0 < nch)
            def _():
                pack_rows(rows0, pk)
                wb_start(c0)

            return carry

        lax.fori_loop(0, n_ss + 1, body, 0)

    return gather


def kernel(x, neighbors, max_iter):
    n = x.shape[0]
    e = neighbors.shape[0]

    xn, xni = pl.pallas_call(
        _normalize_body,
        out_shape=(jax.ShapeDtypeStruct((n, D), jnp.float32),
                   jax.ShapeDtypeStruct((n + PAD, D), jnp.int32)),
    )(x)

    n_chunks = 5
    cn = n // n_chunks
    ce = cn * M
    blk = 400
    grid = cn // blk
    mi = jnp.reshape(jnp.asarray(max_iter, jnp.int32), (1,))
    gather = _make_gather(n + PAD, ce, 2)

    outs = []
    for i in range(n_chunks):
        nbr_i = lax.slice_in_dim(neighbors, i * ce, (i + 1) * ce)
        z_i = gather(xni, nbr_i)
        base = i * grid
        u_i = pl.pallas_call(
            _routing_body,
            grid=(grid,),
            in_specs=[
                pl.BlockSpec(memory_space=pltpu.SMEM),
                pl.BlockSpec((blk * M, D // 2), lambda j: (j, 0)),
                pl.BlockSpec((blk, D), lambda j, base=base: (base + j, 0)),
            ],
            out_specs=pl.BlockSpec((blk, D), lambda j: (j, 0)),
            out_shape=jax.ShapeDtypeStruct((cn, D), jnp.float32),
        )(mi, z_i, xn)
        outs.append(u_i)
    return jnp.concatenate(outs, axis=0)


# speedup vs baseline: 1.0639x; 1.0636x over previous
import functools

import jax
import jax.numpy as jnp
from jax import lax
from jax.experimental import pallas as pl
from jax.experimental.pallas import tpu as pltpu
from jax.experimental.pallas import tpu_sc as plsc

D = 128
K = 8
DD = D // K
M = 32
PAD = 8


def _seg_matrices(dtype=jnp.float32):
    lane = lax.broadcasted_iota(jnp.int32, (D, K), 0)
    cap = lax.broadcasted_iota(jnp.int32, (D, K), 1)
    s = (lane // DD == cap).astype(dtype)
    lane_t = lax.broadcasted_iota(jnp.int32, (K, D), 1)
    cap_t = lax.broadcasted_iota(jnp.int32, (K, D), 0)
    st = (lane_t // DD == cap_t).astype(dtype)
    return s, st


def _normalize_body(x_ref, o_ref):
    x = x_ref[...]
    n = x.shape[0]
    s, st = _seg_matrices()
    ss = lax.dot_general(x * x, s, (((1,), (0,)), ((), ())),
                         preferred_element_type=jnp.float32)
    den = jnp.maximum(jnp.sqrt(ss), 1e-12)
    inv = lax.dot_general(1.0 / den, st, (((1,), (0,)), ((), ())),
                          preferred_element_type=jnp.float32)
    o_ref[pl.ds(0, n), :] = x * inv
    o_ref[pl.ds(n, PAD), :] = jnp.zeros((PAD, D), jnp.float32)


def _routing_body(mi_ref, z_ref, x_ref, o_ref):
    zb = z_ref[...]
    xb = x_ref[...]
    b = xb.shape[0]
    s, st = _seg_matrices()

    def seg_sum(t):
        return lax.dot_general(t, s, (((1,), (0,)), ((), ())),
                               preferred_element_type=jnp.float32)

    def expand(t):
        return lax.dot_general(t, st, (((1,), (0,)), ((), ())),
                               preferred_element_type=jnp.float32)

    ones_kd = jnp.ones((K, D), jnp.float32)

    def body(_, u):
        ss = seg_sum(u * u)
        den = jnp.maximum(jnp.sqrt(ss), 1e-12)
        un = u * expand(1.0 / den)
        unb = jnp.broadcast_to(un.reshape(b, 1, D), (b, M, D)).reshape(b * M, D)
        logits = seg_sum(zb * unb)
        e = jnp.exp(logits)
        num = expand(e)
        dsum = lax.dot_general(e, ones_kd, (((1,), (0,)), ((), ())),
                               preferred_element_type=jnp.float32)
        w = zb * (num / dsum)
        return jnp.sum(w.reshape(b, M, D), axis=1) + xb

    u0 = jnp.zeros((b, D), jnp.float32)
    o_ref[...] = lax.fori_loop(0, mi_ref[0], body, u0)


def _make_gather(n_rows, e, kc):
    mesh = plsc.VectorSubcoreMesh(core_axis_name="c", subcore_axis_name="s")
    nw = 32
    ch = kc * 128
    nch = e // ch
    nb = 3
    n_ss = pl.cdiv(nch, nb * nw)

    @functools.partial(
        pl.kernel, mesh=mesh,
        out_type=jax.ShapeDtypeStruct((e, D), jnp.float32),
        scratch_types=(
            [pltpu.VMEM((kc, 128), jnp.int32) for _ in range(nb)]
            + [pltpu.VMEM((ch, D), jnp.float32) for _ in range(nb)]
            + [pltpu.SemaphoreType.DMA for _ in range(2 * nb)]
        ),
    )
    def gather(table_hbm, nbr_hbm, out_hbm,
               idx0, idx1, idx2, rows0, rows1, rows2,
               g0, g1, g2, w0, w1, w2):
        idx = (idx0, idx1, idx2)
        rows = (rows0, rows1, rows2)
        g = (g0, g1, g2)
        w = (w0, w1, w2)
        wid = lax.axis_index("s") * 2 + lax.axis_index("c")

        def fetch(c, b):
            for j in range(kc):
                pltpu.sync_copy(nbr_hbm.at[pl.ds(c * ch + j * 128, 128)],
                                idx[b].at[j])
            for j in range(kc):
                pltpu.async_copy(
                    table_hbm.at[idx[b].at[j]],
                    rows[b].at[pl.ds(j * 128, 128)], g[b])

        def fetch_wait(b):
            for j in range(kc):
                pltpu.make_async_copy(
                    table_hbm.at[idx[b].at[j]],
                    rows[b].at[pl.ds(j * 128, 128)], g[b]).wait()

        def wb_start(c, b):
            pltpu.async_copy(rows[b], out_hbm.at[pl.ds(c * ch, ch)], w[b])

        def wb_wait(c, b):
            pltpu.make_async_copy(
                rows[b], out_hbm.at[pl.ds(c * ch, ch)], w[b]).wait()

        def body(ss, carry):
            base = ss * nb * nw + wid
            for b in range(nb):
                c = base + b * nw
                cp = c - nw
                cw = c - nb * nw

                @pl.when(jnp.logical_and(cw >= 0, cw < nch))
                def _(cw=cw, b=b):
                    wb_wait(cw, b)

                @pl.when(c < nch)
                def _(c=c, b=b):
                    fetch(c, b)

                @pl.when(jnp.logical_and(cp >= 0, cp < nch))
                def _(cp=cp, b=b):
                    pb = (b - 1) % nb
                    fetch_wait(pb)
                    wb_start(cp, pb)

            return carry

        lax.fori_loop(0, n_ss + 1, body, 0)

    return gather


def kernel(x, neighbors, max_iter):
    n = x.shape[0]
    e = neighbors.shape[0]

    xn = pl.pallas_call(
        _normalize_body,
        out_shape=jax.ShapeDtypeStruct((n + PAD, D), jnp.float32),
    )(x)

    n_chunks = 5
    cn = n // n_chunks
    ce = cn * M
    blk = 400
    grid = cn // blk
    mi = jnp.reshape(jnp.asarray(max_iter, jnp.int32), (1,))
    gather = _make_gather(n + PAD, ce, 2)

    outs = []
    for i in range(n_chunks):
        nbr_i = lax.slice_in_dim(neighbors, i * ce, (i + 1) * ce)
        z_i = gather(xn, nbr_i)
        base = i * grid
        u_i = pl.pallas_call(
            _routing_body,
            grid=(grid,),
            in_specs=[
                pl.BlockSpec(memory_space=pltpu.SMEM),
                pl.BlockSpec((blk * M, D), lambda j: (j, 0)),
                pl.BlockSpec((blk, D), lambda j, base=base: (base + j, 0)),
            ],
            out_specs=pl.BlockSpec((blk, D), lambda j: (j, 0)),
            out_shape=jax.ShapeDtypeStruct((cn, D), jnp.float32),
        )(mi, z_i, xn)
        outs.append(u_i)
    return jnp.concatenate(outs, axis=0)
